# baseline (speedup 1.0000x reference)
import functools

import jax
import jax.numpy as jnp
from jax import lax
from jax.experimental import pallas as pl
from jax.experimental.pallas import tpu as pltpu
from jax.experimental.pallas import tpu_sc as plsc

NC = 2
NS = 16
NW = NC * NS

D = 64
V = 1000000
NBANDS = V // 128
TAIL = V - NBANDS * 128
NBMAX = -(-NBANDS // NW)

CHUNK = 512
SUB = 128
NSUB = CHUNK // SUB
NBUF = 2


@jax.jit
def _transpose_table(wt):
    mesh = plsc.VectorSubcoreMesh(core_axis_name="c", subcore_axis_name="s")

    @functools.partial(
        pl.kernel,
        mesh=mesh,
        out_type=jax.ShapeDtypeStruct((V // 2, 2 * D), jnp.float32),
        scratch_types=[
            pltpu.VMEM((D, 129), jnp.float32),
            pltpu.VMEM((D, 129), jnp.float32),
            pltpu.VMEM((64, 2 * D), jnp.float32),
            pltpu.VMEM((64, 2 * D), jnp.float32),
            pltpu.VMEM((D, 64), jnp.float32),
        ]
        + [pltpu.SemaphoreType.DMA] * (2 * NBUF),
        compiler_params=pltpu.CompilerParams(
            use_tc_tiling_on_sc=True, needs_layout_passes=False
        ),
    )
    def body(wt_hbm, out_hbm, bi0, bi1, bo0, bo1, buf_tail, *sems):
        buf_in = [bi0, bi1]
        buf_out = [bo0, bo1]
        insem = sems[0:NBUF]
        outsem = sems[NBUF : 2 * NBUF]
        wid = lax.axis_index("s") * NC + lax.axis_index("c")
        nb_w = 244 + jnp.where(wid < NBANDS - 244 * NW, 1, 0)
        iota16 = lax.iota(jnp.int32, 16)

        def transpose_block(b_in, b_out, ncols):
            col_rows = [iota16 + c0 for c0 in range(0, D, 16)]

            def pair(p, carry):
                rl0 = p * 2
                vals = []
                for dr in range(2):
                    cols = jnp.broadcast_to(rl0 + dr, (16,)).astype(jnp.int32)
                    for g in range(D // 16):
                        vals.append(plsc.load_gather(b_in, [col_rows[g], cols]))
                for k in range(2 * (D // 16)):
                    dr, g = divmod(k, D // 16)
                    b_out[p, pl.ds(dr * D + g * 16, 16)] = vals[k]
                return carry

            lax.fori_loop(0, ncols // 2, pair, 0)

        pltpu.async_copy(
            wt_hbm.at[:, pl.ds(wid * 128, 128)], buf_in[0].at[:, pl.ds(0, 128)], insem[0]
        )

        def step2(g, carry):
            for s in range(NBUF):
                i = g * NBUF + s

                @pl.when(i < nb_w)
                def _():
                    b = wid + i * NW
                    col0 = b * 128
                    pltpu.make_async_copy(
                        wt_hbm.at[:, pl.ds(col0, 128)], buf_in[s].at[:, pl.ds(0, 128)], insem[s]
                    ).wait()

                    @pl.when(i + 1 < nb_w)
                    def _():
                        pltpu.async_copy(
                            wt_hbm.at[:, pl.ds(col0 + NW * 128, 128)],
                            buf_in[(s + 1) % NBUF].at[:, pl.ds(0, 128)],
                            insem[(s + 1) % NBUF],
                        )

                    @pl.when(i >= NBUF)
                    def _():
                        pltpu.make_async_copy(
                            buf_out[s],
                            out_hbm.at[pl.ds(0, 64)],
                            outsem[s],
                        ).wait()

                    transpose_block(buf_in[s], buf_out[s], 128)
                    pltpu.async_copy(
                        buf_out[s],
                        out_hbm.at[pl.ds(b * 64, 64)],
                        outsem[s],
                    )

            return carry

        lax.fori_loop(0, (NBMAX + NBUF - 1) // NBUF, step2, 0)
        for s in range(NBUF):
            pltpu.make_async_copy(
                buf_out[s], out_hbm.at[pl.ds(0, 64)], outsem[s]
            ).wait()

        @pl.when(wid == 0)
        def _():
            pltpu.sync_copy(wt_hbm.at[:, pl.ds(NBANDS * 128, TAIL)], buf_tail)
            col_rows_t = [iota16 + c0 for c0 in range(0, D, 16)]

            def tpair(p, carry):
                rl0 = p * 2
                vals = []
                for dr in range(2):
                    cols = jnp.broadcast_to(rl0 + dr, (16,)).astype(jnp.int32)
                    for g in range(D // 16):
                        vals.append(
                            plsc.load_gather(buf_tail, [col_rows_t[g], cols])
                        )
                for k in range(2 * (D // 16)):
                    dr, g = divmod(k, D // 16)
                    buf_out[0][p, pl.ds(dr * D + g * 16, 16)] = vals[k]
                return carry

            lax.fori_loop(0, TAIL // 2, tpair, 0)
            pltpu.sync_copy(
                buf_out[0].at[pl.ds(0, TAIL // 2)],
                out_hbm.at[pl.ds(NBANDS * 64, TAIL // 2)],
            )

    return body(wt)


@functools.partial(jax.jit, static_argnums=(2,))
def _emb_gather(idx_flat, table, bpw):
    nch = bpw // CHUNK
    assert nch % NBUF == 0
    mesh = plsc.VectorSubcoreMesh(core_axis_name="c", subcore_axis_name="s")

    @functools.partial(
        pl.kernel,
        mesh=mesh,
        out_type=jax.ShapeDtypeStruct((idx_flat.shape[0], D), jnp.float32),
        scratch_types=[
            pltpu.VMEM((NBUF, CHUNK), jnp.int32),
            pltpu.VMEM((NBUF, CHUNK, D), jnp.float32),
        ]
        + [pltpu.SemaphoreType.DMA] * (3 * NBUF),
        compiler_params=pltpu.CompilerParams(use_tc_tiling_on_sc=False),
    )
    def body(idx_hbm, w_hbm, out_hbm, idx_v, rows_v, *sems):
        gsem = sems[0:NBUF]
        osem = sems[NBUF : 2 * NBUF]
        isem = sems[2 * NBUF : 3 * NBUF]
        wid = lax.axis_index("s") * NC + lax.axis_index("c")
        base = wid * bpw

        pltpu.async_copy(idx_hbm.at[pl.ds(base, CHUNK)], idx_v.at[0], isem[0])

        def step(g, carry):
            for b in range(NBUF):
                c = g * NBUF + b
                off = base + c * CHUNK
                pltpu.make_async_copy(
                    idx_hbm.at[pl.ds(off, CHUNK)], idx_v.at[b], isem[b]
                ).wait()

                @pl.when(c >= NBUF)
                def _():
                    pltpu.make_async_copy(
                        rows_v.at[b], out_hbm.at[pl.ds(off, CHUNK)], osem[b]
                    ).wait()

                copies = [
                    pltpu.async_copy(
                        w_hbm.at[idx_v.at[b].at[pl.ds(j * SUB, SUB)]],
                        rows_v.at[b].at[pl.ds(j * SUB, SUB)],
                        gsem[b],
                    )
                    for j in range(NSUB)
                ]
                nb = (b + 1) % NBUF

                @pl.when(c + 1 < nch)
                def _():
                    pltpu.async_copy(
                        idx_hbm.at[pl.ds(off + CHUNK, CHUNK)], idx_v.at[nb], isem[nb]
                    )

                for cp in copies:
                    cp.wait()
                pltpu.async_copy(rows_v.at[b], out_hbm.at[pl.ds(off, CHUNK)], osem[b])
            return carry

        lax.fori_loop(0, nch // NBUF, step, 0)
        for b in range(NBUF):
            pltpu.make_async_copy(
                rows_v.at[b], out_hbm.at[pl.ds(base, CHUNK)], osem[b]
            ).wait()

    return body(idx_flat, table)


def kernel(input, W):
    idx = input.reshape(-1).astype(jnp.int32)
    w_lin = _transpose_table(W.T).reshape(V, D)
    bpw = idx.shape[0] // NW
    out = _emb_gather(idx, w_lin, bpw)
    return out.reshape(input.shape + (W.shape[1],))


# speedup vs baseline: 1.4141x; 1.4141x over previous
import functools

import jax
import jax.numpy as jnp
from jax import lax
from jax.experimental import pallas as pl
from jax.experimental.pallas import tpu as pltpu
from jax.experimental.pallas import tpu_sc as plsc

NC = 2
NS = 16
NW = NC * NS

D = 64
DP = 128
CHUNK = 512
SUB = 128
NSUB = CHUNK // SUB
NBUF = 2


@functools.partial(jax.jit, static_argnums=(2,))
def _emb_lookup(idx_flat, table, bpw):
    nch = bpw // CHUNK
    assert nch % NBUF == 0
    mesh = plsc.VectorSubcoreMesh(core_axis_name="c", subcore_axis_name="s")

    @functools.partial(
        pl.kernel,
        mesh=mesh,
        out_type=jax.ShapeDtypeStruct((idx_flat.shape[0], DP), jnp.float32),
        scratch_types=[
            pltpu.VMEM((NBUF, CHUNK), jnp.int32),
            pltpu.VMEM((NBUF, CHUNK, D), jnp.float32),
        ]
        + [pltpu.SemaphoreType.DMA] * (3 * NBUF),
        compiler_params=pltpu.CompilerParams(use_tc_tiling_on_sc=False),
    )
    def body(idx_hbm, w_hbm, out_hbm, idx_v, rows_v, *sems):
        gsem = sems[0:NBUF]
        osem = sems[NBUF : 2 * NBUF]
        isem = sems[2 * NBUF : 3 * NBUF]
        wid = lax.axis_index("s") * NC + lax.axis_index("c")
        base = wid * bpw

        pltpu.async_copy(idx_hbm.at[pl.ds(base, CHUNK)], idx_v.at[0], isem[0])

        def step(g, carry):
            for b in range(NBUF):
                c = g * NBUF + b
                off = base + c * CHUNK
                pltpu.make_async_copy(
                    idx_hbm.at[pl.ds(off, CHUNK)], idx_v.at[b], isem[b]
                ).wait()
                @pl.when(c >= NBUF)
                def _():
                    pltpu.make_async_copy(
                        rows_v.at[b],
                        out_hbm.at[pl.ds(off, CHUNK), pl.ds(0, D)],
                        osem[b],
                    ).wait()
                copies = [
                    pltpu.async_copy(
                        w_hbm.at[idx_v.at[b].at[pl.ds(j * SUB, SUB)]],
                        rows_v.at[b].at[pl.ds(j * SUB, SUB)],
                        gsem[b],
                    )
                    for j in range(NSUB)
                ]
                nb = (b + 1) % NBUF

                @pl.when(c + 1 < nch)
                def _():
                    pltpu.async_copy(
                        idx_hbm.at[pl.ds(off + CHUNK, CHUNK)], idx_v.at[nb], isem[nb]
                    )

                for cp in copies:
                    cp.wait()
                pltpu.async_copy(
                    rows_v.at[b],
                    out_hbm.at[pl.ds(off, CHUNK), pl.ds(0, D)],
                    osem[b],
                )
            return carry

        lax.fori_loop(0, nch // NBUF, step, 0)
        for b in range(NBUF):
            pltpu.make_async_copy(
                rows_v.at[b],
                out_hbm.at[pl.ds(base, CHUNK), pl.ds(0, D)],
                osem[b],
            ).wait()

    return body(idx_flat, table)


def kernel(input, W):
    idx = input.reshape(-1).astype(jnp.int32)
    bpw = idx.shape[0] // NW
    out = _emb_lookup(idx, W, bpw)
    return out[:, :D].reshape(input.shape + (W.shape[1],))


# speedup vs baseline: 1.4833x; 1.0489x over previous
import functools

import jax
import jax.numpy as jnp
from jax import lax
from jax.experimental import pallas as pl
from jax.experimental.pallas import tpu as pltpu
from jax.experimental.pallas import tpu_sc as plsc

NC = 2
NS = 16
NW = NC * NS

D = 64
CHUNK = 512
SUB = 128
NSUB = CHUNK // SUB
NBUF = 2


@functools.partial(jax.jit, static_argnums=(2,))
def _emb_lookup(idx_flat, table, bpw):
    nch = bpw // CHUNK
    assert nch % NBUF == 0
    mesh = plsc.VectorSubcoreMesh(core_axis_name="c", subcore_axis_name="s")

    @functools.partial(
        pl.kernel,
        mesh=mesh,
        out_type=jax.ShapeDtypeStruct((idx_flat.shape[0], D), jnp.float32),
        scratch_types=[
            pltpu.VMEM((NBUF, CHUNK), jnp.int32),
            pltpu.VMEM((NBUF, CHUNK, D), jnp.float32),
        ]
        + [pltpu.SemaphoreType.DMA] * (3 * NBUF),
        compiler_params=pltpu.CompilerParams(use_tc_tiling_on_sc=False),
    )
    def body(idx_hbm, w_hbm, out_hbm, idx_v, rows_v, *sems):
        gsem = sems[0:NBUF]
        osem = sems[NBUF : 2 * NBUF]
        isem = sems[2 * NBUF : 3 * NBUF]
        wid = lax.axis_index("s") * NC + lax.axis_index("c")
        base = wid * bpw

        pltpu.async_copy(idx_hbm.at[pl.ds(base, CHUNK)], idx_v.at[0], isem[0])

        def step(g, carry):
            for b in range(NBUF):
                c = g * NBUF + b
                off = base + c * CHUNK
                pltpu.make_async_copy(
                    idx_hbm.at[pl.ds(off, CHUNK)], idx_v.at[b], isem[b]
                ).wait()
                @pl.when(c >= NBUF)
                def _():
                    pltpu.make_async_copy(
                        rows_v.at[b], out_hbm.at[pl.ds(off, CHUNK)], osem[b]
                    ).wait()
                copies = [
                    pltpu.async_copy(
                        w_hbm.at[idx_v.at[b].at[pl.ds(j * SUB, SUB)]],
                        rows_v.at[b].at[pl.ds(j * SUB, SUB)],
                        gsem[b],
                    )
                    for j in range(NSUB)
                ]
                nb = (b + 1) % NBUF

                @pl.when(c + 1 < nch)
                def _():
                    pltpu.async_copy(
                        idx_hbm.at[pl.ds(off + CHUNK, CHUNK)], idx_v.at[nb], isem[nb]
                    )

                for cp in copies:
                    cp.wait()
                pltpu.async_copy(rows_v.at[b], out_hbm.at[pl.ds(off, CHUNK)], osem[b])
            return carry

        lax.fori_loop(0, nch // NBUF, step, 0)
        for b in range(NBUF):
            pltpu.make_async_copy(
                rows_v.at[b], out_hbm.at[pl.ds(base, CHUNK)], osem[b]
            ).wait()

    return body(idx_flat, table)


def kernel(input, W):
    idx = input.reshape(-1).astype(jnp.int32)
    bpw = idx.shape[0] // NW
    out = _emb_lookup(idx, W, bpw)
    return out.reshape(input.shape + (W.shape[1],))
